# baseline (speedup 1.0000x reference)
import functools

import jax
import jax.numpy as jnp
from jax import lax
from jax.experimental import pallas as pl
from jax.experimental.pallas import tpu as pltpu
from jax.experimental.pallas import tpu_sc as plsc

_NC = 2
_NS = 16
_NW = _NC * _NS


@functools.lru_cache(maxsize=None)
def _make_gather(n_tokens: int, hidden: int):
    per_w = n_tokens // _NW
    mesh = plsc.VectorSubcoreMesh(core_axis_name="c", subcore_axis_name="s")

    @functools.partial(
        pl.kernel,
        out_type=jax.ShapeDtypeStruct((n_tokens, hidden), jnp.float32),
        mesh=mesh,
        scratch_types=[
            pltpu.VMEM((per_w,), jnp.int32),
            pltpu.SemaphoreType.DMA,
        ],
    )
    def gather_kernel(table_hbm, ids_hbm, out_hbm, idx_v, sem):
        wid = lax.axis_index("s") * _NC + lax.axis_index("c")
        base = pl.multiple_of(wid * per_w, 8)
        pltpu.sync_copy(ids_hbm.at[pl.ds(base, per_w)], idx_v)

        def body(g, carry):
            v = idx_v[pl.ds(pl.multiple_of(g * 16, 8), 16)]
            for k in range(16):
                r = v[k]
                pltpu.make_async_copy(
                    table_hbm.at[pl.ds(r, 1)],
                    out_hbm.at[pl.ds(base + g * 16 + k, 1)],
                    sem,
                ).start()
            for k in range(16):
                pltpu.make_async_copy(
                    table_hbm.at[pl.ds(0, 1)],
                    out_hbm.at[pl.ds(base, 1)],
                    sem,
                ).wait()
            return carry

        lax.fori_loop(0, per_w // 16, body, 0)

    return gather_kernel


def kernel(input_ids, embed_tokens):
    batch, seq = input_ids.shape
    _, hidden = embed_tokens.shape
    ids_flat = input_ids.reshape(-1).astype(jnp.int32)
    rows = _make_gather(batch * seq, hidden)(embed_tokens, ids_flat)
    inputs_embeds = rows.reshape(batch, seq, hidden)
    cache_position = jnp.arange(seq, dtype=jnp.int32)
    position_ids = cache_position[None, :]
    return (inputs_embeds, position_ids, cache_position)


# speedup vs baseline: 39.1847x; 39.1847x over previous
import functools

import jax
import jax.numpy as jnp
from jax import lax
from jax.experimental import pallas as pl
from jax.experimental.pallas import tpu as pltpu
from jax.experimental.pallas import tpu_sc as plsc

_NC = 2
_NS = 16
_NW = _NC * _NS

_CH = 16
_NBUF = 3


@functools.lru_cache(maxsize=None)
def _make_gather(n_tokens: int, hidden: int):
    per_w = n_tokens // _NW
    nch = per_w // _CH
    assert nch % _NBUF == 1
    mesh = plsc.VectorSubcoreMesh(core_axis_name="c", subcore_axis_name="s")

    @functools.partial(
        pl.kernel,
        out_type=jax.ShapeDtypeStruct((n_tokens, hidden), jnp.float32),
        mesh=mesh,
        scratch_types=[
            pltpu.VMEM((per_w,), jnp.int32),
            pltpu.VMEM((_NBUF, _CH, hidden), jnp.float32),
            pltpu.SemaphoreType.DMA,
            pltpu.SemaphoreType.DMA,
            pltpu.SemaphoreType.DMA,
            pltpu.SemaphoreType.DMA,
            pltpu.SemaphoreType.DMA,
            pltpu.SemaphoreType.DMA,
        ],
    )
    def gather_kernel(table_hbm, ids_hbm, out_hbm, idx_v, rows_v,
                      g0, g1, g2, w0, w1, w2):
        gsem = (g0, g1, g2)
        wsem = (w0, w1, w2)
        wid = lax.axis_index("s") * _NC + lax.axis_index("c")
        base = pl.multiple_of(wid * per_w, 8)

        pltpu.sync_copy(ids_hbm.at[pl.ds(base, per_w)], idx_v)

        def chunk_idx(c):
            return idx_v.at[pl.ds(pl.multiple_of(c * _CH, 8), _CH)]

        def gather_copy(c, b):
            return pltpu.make_async_copy(
                table_hbm.at[chunk_idx(c)], rows_v.at[b], gsem[b])

        def write_copy(c, b):
            return pltpu.make_async_copy(
                rows_v.at[b],
                out_hbm.at[pl.ds(pl.multiple_of(base + c * _CH, 8), _CH)],
                wsem[b])

        for b in range(_NBUF):
            gather_copy(b, b).start()
        gather_copy(0, 0).wait()
        write_copy(0, 0).start()

        def body(i, carry):
            c0 = 1 + i * _NBUF
            for j in range(_NBUF):
                c = c0 + j
                bp = j
                b = (j + 1) % 3
                write_copy(c - 1, bp).wait()
                gather_copy(c + 2, bp).start()
                gather_copy(c, b).wait()
                write_copy(c, b).start()
            return carry

        lax.fori_loop(0, (nch - 4) // _NBUF, body, 0)

        c = nch - 3
        write_copy(c - 1, 0).wait()
        gather_copy(c + 2, 0).start()
        gather_copy(c, 1).wait()
        write_copy(c, 1).start()

        c = nch - 2
        write_copy(c - 1, 1).wait()
        gather_copy(c, 2).wait()
        write_copy(c, 2).start()

        c = nch - 1
        write_copy(c - 1, 2).wait()
        gather_copy(c, 0).wait()
        write_copy(c, 0).start()
        write_copy(c, 0).wait()

    return gather_kernel


def kernel(input_ids, embed_tokens):
    batch, seq = input_ids.shape
    _, hidden = embed_tokens.shape
    ids_flat = input_ids.reshape(-1).astype(jnp.int32)
    rows = _make_gather(batch * seq, hidden)(embed_tokens, ids_flat)
    inputs_embeds = rows.reshape(batch, seq, hidden)
    cache_position = jnp.arange(seq, dtype=jnp.int32)
    position_ids = cache_position[None, :]
    return (inputs_embeds, position_ids, cache_position)
